# baseline (speedup 1.0000x reference)
import functools

import jax
import jax.numpy as jnp
from jax import lax
from jax.experimental import pallas as pl
from jax.experimental.pallas import tpu as pltpu
from jax.experimental.pallas import tpu_sc as plsc

B = 16384
NC, NS, L = 2, 16, 16
NW = NC * NS
RPW = B // NW
CHUNKS = RPW // L
NF = 15
NFP = 16
BLK = 2048
NB = B // BLK

O_ER, O_EH, O_HS, O_FS, O_HP, O_CO = 0, 6, 594, 614, 622, 654
TAB = 688


_mesh = plsc.VectorSubcoreMesh(core_axis_name="c", subcore_axis_name="s",
                               num_cores=NC, num_subcores=NS)


@functools.partial(
    pl.kernel,
    out_type=jax.ShapeDtypeStruct((NB * NFP * BLK,), jnp.float32),
    mesh=_mesh,
    compiler_params=pltpu.CompilerParams(needs_layout_passes=False),
    scratch_types=[
        pltpu.VMEM((RPW,), jnp.int32),
        pltpu.VMEM((RPW,), jnp.int32),
        pltpu.VMEM((RPW,), jnp.int32),
        pltpu.VMEM((RPW,), jnp.int32),
        pltpu.VMEM((RPW,), jnp.int32),
        pltpu.VMEM((RPW,), jnp.int32),
        pltpu.VMEM((RPW,), jnp.int32),
        pltpu.VMEM((RPW,), jnp.int32),
        pltpu.VMEM((TAB,), jnp.float32),
        pltpu.VMEM((NFP * RPW,), jnp.float32),
        pltpu.SemaphoreType.DMA,
    ],
)
def _sc_gather(rp_h, cp_h, hand_h, hs_h, fs_h, hp_h, fp_h, co_h, tab_h, out_h,
               rp_b, cp_b, hand_b, hs_b, fs_b, hp_b, fp_b, co_b,
               tab_v, out_v, sem):
    wid = lax.axis_index("s") * NC + lax.axis_index("c")
    base = wid * RPW

    copies = [
        pltpu.async_copy(src.at[pl.ds(base, RPW)], dst, sem)
        for src, dst in ((rp_h, rp_b), (cp_h, cp_b), (hand_h, hand_b),
                         (hs_h, hs_b), (fs_h, fs_b), (hp_h, hp_b),
                         (fp_h, fp_b), (co_h, co_b))
    ]
    copies.append(pltpu.async_copy(tab_h, tab_v, sem))
    for c in copies:
        c.wait()

    @plsc.parallel_loop(0, CHUNKS, 1, unroll=4)
    def chunk(c):
        s = c * L
        rp_v = rp_b[pl.ds(s, L)]
        cp_v = cp_b[pl.ds(s, L)]
        hand_v = hand_b[pl.ds(s, L)]
        hs_v = hs_b[pl.ds(s, L)]
        fs_v = fs_b[pl.ds(s, L)]
        hp_v = hp_b[pl.ds(s, L)]
        fp_v = fp_b[pl.ds(s, L)]
        co_v = co_b[pl.ds(s, L)]

        feats = []
        feats.append(plsc.load_gather(tab_v, [rp_v]))
        feats.append(plsc.load_gather(tab_v, [cp_v]))
        for src, width, vlen, off in (
                (hand_v, 3, 196, O_EH), (hs_v, 2, 10, O_HS),
                (fs_v, 2, 4, O_FS), (hp_v, 2, 16, O_HP),
                (fp_v, 2, 16, O_HP), (co_v, 2, 10, O_CO)):
            idx = src + off
            for w in range(width):
                feats.append(plsc.load_gather(tab_v, [idx]))
                if w < width - 1:
                    idx = idx + vlen
        for k in range(NF):
            out_v[pl.ds(k * RPW + s, L)] = feats[k]

    j = wid // 4
    col = (wid % 4) * RPW
    out_copies = [
        pltpu.async_copy(out_v.at[pl.ds(k * RPW, RPW)],
                         out_h.at[pl.ds((j * NFP + k) * BLK + col, RPW)], sem)
        for k in range(NF)
    ]
    for c in out_copies:
        c.wait()


def _post_body(f_ref, ct_ref, w1t_ref, b1_ref, w2t_ref, b2_ref, out_ref):
    w1t = w1t_ref[...]
    b1 = b1_ref[...]
    w2t = w2t_ref[...]
    b2 = b2_ref[...]
    for j in range(NB):
        ct_t = ct_ref[j * 16:(j + 1) * 16, :]
        fk = [f_ref[(j * NFP + k) * 16:(j * NFP + k + 1) * 16, :]
              for k in range(NF)]
        rs = []
        for d in range(9):
            h = ct_t * w1t[d, 0] + b1[0, d]
            for k in range(NF):
                h = h + fk[k] * w1t[d, 1 + k]
            rs.append(jnp.maximum(h, 0.0))
        for e in range(3):
            o = rs[0] * w2t[e, 0] + b2[0, e]
            for d in range(1, 9):
                o = o + rs[d] * w2t[e, d]
            out_ref[(j * 3 + e) * 16:(j * 3 + e + 1) * 16, :] = o


_post = pl.pallas_call(
    _post_body,
    out_shape=jax.ShapeDtypeStruct((NB * 3 * 16, 128), jnp.float32),
)


def kernel(raiser_pos, caller_pos, checked_to, hand, hands_strength,
           flops_strength, hand_pot, flop_pot, cards_ord,
           emb_raiser, emb_hand, emb_hs, emb_fs, emb_hp, emb_ord,
           W1, b1, W2, b2):
    f32 = jnp.float32
    i32 = jnp.int32

    def _placed(t, off, n):
        return jnp.pad(t.T.astype(f32).reshape(n), (off, TAB - off - n))

    tab = (_placed(emb_raiser, O_ER, 6) + _placed(emb_hand, O_EH, 588)
           + _placed(emb_hs, O_HS, 20) + _placed(emb_fs, O_FS, 8)
           + _placed(emb_hp, O_HP, 32) + _placed(emb_ord, O_CO, 20))

    feats = _sc_gather(raiser_pos.astype(i32), caller_pos.astype(i32),
                       hand.astype(i32), hands_strength.astype(i32),
                       flops_strength.astype(i32), hand_pot.astype(i32),
                       flop_pot.astype(i32), cards_ord.astype(i32), tab)

    out_t = _post(feats.reshape(NB * NFP * 16, 128),
                  checked_to.astype(f32).reshape(NB * 16, 128),
                  W1.astype(f32).T, b1.astype(f32).reshape(1, 9),
                  W2.astype(f32).T, b2.astype(f32).reshape(1, 3))

    return (out_t.reshape(NB, 3, 16, 128).transpose(0, 2, 3, 1)
            .reshape(B, 3))
